# Initial kernel scaffold; baseline (speedup 1.0000x reference)
import jax
import jax.numpy as jnp
from jax.experimental import pallas as pl


def kernel(system_features, batch_index):
    raise NotImplementedError("write your pallas kernel here")


# speedup vs baseline: 1.2415x; 1.2415x over previous
import functools

import jax
import jax.numpy as jnp
from jax import lax
from jax.experimental import pallas as pl
from jax.experimental.pallas import tpu as pltpu
from jax.experimental.pallas import tpu_sc as plsc

NC = 2
NS = 16
NW = NC * NS
CH = 128


@functools.lru_cache(maxsize=None)
def _build(V, D, B):
    assert B % (NW * CH) == 0
    b_per_w = B // NW
    n_ch = b_per_w // CH
    mesh = plsc.VectorSubcoreMesh(core_axis_name="c", subcore_axis_name="s")

    @functools.partial(
        pl.kernel,
        out_type=jax.ShapeDtypeStruct((B, D), jnp.float32),
        mesh=mesh,
        scratch_types=[
            pltpu.VMEM((n_ch, CH), jnp.int32),
            pltpu.VMEM((CH, D), jnp.float32),
            pltpu.SemaphoreType.DMA,
        ],
    )
    def gather_kernel(table_hbm, idx_hbm, out_hbm, idx_v, rows_v, sem):
        wid = lax.axis_index("s") * NC + lax.axis_index("c")
        pltpu.sync_copy(idx_hbm.at[wid], idx_v)
        base = wid * b_per_w
        for g in range(n_ch):
            pltpu.async_copy(table_hbm.at[idx_v.at[g]], rows_v, sem).wait()
            pltpu.sync_copy(rows_v, out_hbm.at[pl.ds(base + g * CH, CH)])

    return gather_kernel


def kernel(system_features, batch_index):
    V, D = system_features.shape
    (B,) = batch_index.shape
    idx = batch_index.astype(jnp.int32).reshape(NW, B // (NW * CH), CH)
    return _build(V, D, B)(system_features, idx)


# speedup vs baseline: 1.6659x; 1.3419x over previous
import functools

import jax
import jax.numpy as jnp
from jax import lax
from jax.experimental import pallas as pl
from jax.experimental.pallas import tpu as pltpu
from jax.experimental.pallas import tpu_sc as plsc

NC = 2
NS = 16
NW = NC * NS
CH = 128
NBUF = 3


@functools.lru_cache(maxsize=None)
def _build(V, D, B):
    assert B % (NW * CH) == 0
    b_per_w = B // NW
    n_ch = b_per_w // CH
    mesh = plsc.VectorSubcoreMesh(core_axis_name="c", subcore_axis_name="s")

    @functools.partial(
        pl.kernel,
        out_type=jax.ShapeDtypeStruct((B, D), jnp.float32),
        mesh=mesh,
        scratch_types=[
            pltpu.VMEM((n_ch, CH), jnp.int32),
            [pltpu.VMEM((CH, D), jnp.float32) for _ in range(NBUF)],
            [pltpu.SemaphoreType.DMA for _ in range(NBUF)],
            [pltpu.SemaphoreType.DMA for _ in range(NBUF)],
        ],
    )
    def gather_kernel(table_hbm, idx_hbm, out_hbm, idx_v, rows, gsem, osem):
        wid = lax.axis_index("s") * NC + lax.axis_index("c")
        pltpu.sync_copy(idx_hbm.at[wid], idx_v)
        base = wid * b_per_w

        def start_gather(g):
            return pltpu.async_copy(
                table_hbm.at[idx_v.at[g]], rows[g % NBUF], gsem[g % NBUF])

        def start_out(g):
            return pltpu.async_copy(
                rows[g % NBUF], out_hbm.at[pl.ds(base + g * CH, CH)],
                osem[g % NBUF])

        gathers = [None] * n_ch
        outs = [None] * n_ch
        for g in range(min(NBUF, n_ch)):
            gathers[g] = start_gather(g)
        for g in range(n_ch):
            gathers[g].wait()
            outs[g] = start_out(g)
            nxt = g + NBUF
            if nxt < n_ch:
                outs[nxt - NBUF].wait()
                gathers[nxt] = start_gather(nxt)
        for g in range(max(0, n_ch - NBUF), n_ch):
            outs[g].wait()

    return gather_kernel


def kernel(system_features, batch_index):
    V, D = system_features.shape
    (B,) = batch_index.shape
    idx = batch_index.astype(jnp.int32).reshape(NW, B // (NW * CH), CH)
    return _build(V, D, B)(system_features, idx)
